# baseline (speedup 1.0000x reference)
import jax
import jax.numpy as jnp
from jax.experimental import pallas as pl


def _mask_kernel(img_ref, out_ref):
    v = img_ref[...]
    out_ref[...] = v * (v < 10000.0).astype(v.dtype)


def kernel(x, flow_in):
    B, C, H, W = x.shape
    HW = H * W
    flow = flow_in
    grid_h = jnp.broadcast_to(jnp.arange(W, dtype=x.dtype).reshape(1, 1, 1, W), (B, 1, H, W))
    grid_v = jnp.broadcast_to(jnp.arange(H, dtype=x.dtype).reshape(1, 1, H, 1), (B, 1, H, W))
    init_grid = jnp.concatenate([grid_h, grid_v], axis=1)
    coords = (init_grid + flow).reshape(B, 2, HW).transpose(0, 2, 1)
    coords_r = jnp.round(coords).astype(jnp.int32)
    point_values = x.reshape(B, C, HW).transpose(0, 2, 1)
    inb = (coords_r[..., 0] >= 0) & (coords_r[..., 0] < W) & (coords_r[..., 1] >= 0) & (coords_r[..., 1] < H)
    coords_r = coords_r * inb[..., None].astype(coords_r.dtype)
    pvn = 1.0 / (point_values[..., -1] + 1e-08)
    pvn = pvn * (pvn < 10000.0).astype(pvn.dtype)
    lin = coords_r[..., 1] * W + coords_r[..., 0]
    b_idx = jnp.broadcast_to(jnp.arange(B)[:, None], (B, HW))
    maxv = jnp.zeros((B, HW), dtype=pvn.dtype).at[b_idx, lin].max(pvn)
    gathered = jnp.take_along_axis(maxv, lin, axis=1)
    pos = jnp.broadcast_to(jnp.arange(HW, dtype=jnp.int32)[None, :], (B, HW))
    cand = jnp.where((pvn == gathered) & (gathered > 0.0), pos, HW)
    arg = jnp.full((B, HW), HW, dtype=jnp.int32).at[b_idx, lin].min(cand)
    arg_valid = (arg >= 0) & (arg < HW)
    arg_m = (arg * arg_valid.astype(arg.dtype)) % HW
    idx3 = jnp.broadcast_to(arg_m[:, :, None], (B, HW, C))
    max_vals = jnp.take_along_axis(point_values, idx3, axis=1)
    max_vals = jnp.where(arg_valid[:, :, None], max_vals, jnp.float32(100000000.0))
    max_vals = max_vals.at[:, 0, :].set(100000000.0)
    image = max_vals.reshape(B, H, W, C).transpose(0, 3, 1, 2)
    image = pl.pallas_call(
        _mask_kernel,
        out_shape=jax.ShapeDtypeStruct((B, C, H, W), x.dtype),
        grid=(B,),
        in_specs=[pl.BlockSpec((1, C, H, W), lambda b: (b, 0, 0, 0))],
        out_specs=pl.BlockSpec((1, C, H, W), lambda b: (b, 0, 0, 0)),
    )(image)
    return image


# speedup vs baseline: 1.4758x; 1.4758x over previous
import functools

import jax
import jax.numpy as jnp
from jax import lax
from jax.experimental import pallas as pl
from jax.experimental.pallas import tpu as pltpu
from jax.experimental.pallas import tpu_sc as plsc

B, C, H, W = 8, 3, 512, 512
HW = H * W
NW = 32
SHARD = HW // NW
CHUNK = 16384
VPC = CHUNK // 16
ROWS = 128


def _prep_body(flow_ref, depth_ref, lin_ref, pvn_ref):
    r = pl.program_id(1)
    fx = flow_ref[0, 0]
    fy = flow_ref[0, 1]
    gx = lax.broadcasted_iota(jnp.int32, (ROWS, W), 1).astype(jnp.float32)
    gy = lax.broadcasted_iota(jnp.int32, (ROWS, W), 0).astype(jnp.float32) + (r * ROWS).astype(jnp.float32)
    cxf = jnp.round(gx + fx)
    cyf = jnp.round(gy + fy)
    inb = (cxf >= 0) & (cxf < W) & (cyf >= 0) & (cyf < H)
    cx = jnp.clip(cxf, 0, W - 1).astype(jnp.int32)
    cy = jnp.clip(cyf, 0, H - 1).astype(jnp.int32)
    lin_ref[0] = jnp.where(inb, cy * W + cx, 0)
    v = depth_ref[0, 0]
    pvn = 1.0 / (v + 1e-08)
    pvn_ref[0] = pvn * (pvn < 10000.0).astype(jnp.float32)


def _prep(x, flow_in):
    lin, pvn = pl.pallas_call(
        _prep_body,
        out_shape=(
            jax.ShapeDtypeStruct((B, H, W), jnp.int32),
            jax.ShapeDtypeStruct((B, H, W), jnp.float32),
        ),
        grid=(B, H // ROWS),
        in_specs=[
            pl.BlockSpec((1, 2, ROWS, W), lambda b, r: (b, 0, r, 0)),
            pl.BlockSpec((1, 1, ROWS, W), lambda b, r: (b, 2, r, 0)),
        ],
        out_specs=(
            pl.BlockSpec((1, ROWS, W), lambda b, r: (b, r, 0)),
            pl.BlockSpec((1, ROWS, W), lambda b, r: (b, r, 0)),
        ),
    )(flow_in, x)
    return lin.reshape(B * HW), pvn.reshape(B * HW)


def _splat_body(lin_hbm, pvn_hbm, x_hbm, out_hbm,
                lin_v, pvn_v, maxv_fb, argp_fb, idx_v, gath_v, outb_v, sem):
    wid = lax.axis_index("s") * 2 + lax.axis_index("c")
    base = wid * SHARD
    iota = lax.iota(jnp.int32, 16)

    def per_batch(b, _):
        pt_base = b * HW

        def init(i, _):
            maxv_fb[pl.ds(i * 16, 16)] = jnp.zeros((16,), jnp.float32)
            argp_fb[pl.ds(i * 16, 16)] = jnp.full((16,), HW, jnp.int32)
            return 0

        lax.fori_loop(0, SHARD // 16, init, 0)

        def pass1_chunk(ci, _):
            off = pt_base + ci * CHUNK
            pltpu.sync_copy(lin_hbm.at[pl.ds(off, CHUNK)], lin_v)
            pltpu.sync_copy(pvn_hbm.at[pl.ds(off, CHUNK)], pvn_v)

            def vloop(i, _):
                l = lin_v[pl.ds(i * 16, 16)]
                v = pvn_v[pl.ds(i * 16, 16)]
                loc = l - base
                act = (loc >= 0) & (loc < SHARD) & (v > 0.0)
                locs = jnp.clip(loc, 0, SHARD - 1)

                @pl.when(jnp.any(act))
                def _():
                    g = plsc.load_gather(maxv_fb, [locs])
                    need = act & (v > g)

                    def cond(m):
                        return jnp.any(m != 0)

                    def body(m):
                        plsc.store_scatter(maxv_fb, [locs], v, mask=m != 0)
                        g2 = plsc.load_gather(maxv_fb, [locs])
                        return (act & (v > g2)).astype(jnp.int32)

                    lax.while_loop(cond, body, need.astype(jnp.int32))

                return 0

            lax.fori_loop(0, VPC, vloop, 0)
            return 0

        lax.fori_loop(0, HW // CHUNK, pass1_chunk, 0)

        def pass2_chunk(ci, _):
            off = pt_base + ci * CHUNK
            pltpu.sync_copy(lin_hbm.at[pl.ds(off, CHUNK)], lin_v)
            pltpu.sync_copy(pvn_hbm.at[pl.ds(off, CHUNK)], pvn_v)

            def vloop(i, _):
                l = lin_v[pl.ds(i * 16, 16)]
                v = pvn_v[pl.ds(i * 16, 16)]
                loc = l - base
                act = (loc >= 0) & (loc < SHARD) & (v > 0.0)
                locs = jnp.clip(loc, 0, SHARD - 1)
                p = (ci * CHUNK + i * 16) + iota

                @pl.when(jnp.any(act))
                def _():
                    g = plsc.load_gather(maxv_fb, [locs])
                    win = act & (v == g)
                    ga = plsc.load_gather(argp_fb, [locs])
                    need = win & (p < ga)

                    def cond(m):
                        return jnp.any(m != 0)

                    def body(m):
                        plsc.store_scatter(argp_fb, [locs], p, mask=m != 0)
                        ga2 = plsc.load_gather(argp_fb, [locs])
                        return (win & (p < ga2)).astype(jnp.int32)

                    lax.while_loop(cond, body, need.astype(jnp.int32))

                return 0

            lax.fori_loop(0, VPC, vloop, 0)
            return 0

        lax.fori_loop(0, HW // CHUNK, pass2_chunk, 0)

        def mkidx(i, _):
            q = base + i * 16 + iota
            a = argp_fb[pl.ds(i * 16, 16)]
            valid = (a < HW) & (q > 0)
            idx_v[pl.ds(i * 16, 16)] = jnp.where(valid, a, q) + (b * C) * HW
            return 0

        lax.fori_loop(0, SHARD // 16, mkidx, 0)

        for c in range(C):
            if c > 0:
                def bump(i, _):
                    idx_v[pl.ds(i * 16, 16)] = idx_v[pl.ds(i * 16, 16)] + HW
                    return 0

                lax.fori_loop(0, SHARD // 16, bump, 0)
            pltpu.async_copy(x_hbm.at[idx_v], gath_v, sem).wait()

            def emit(i, _):
                q = base + i * 16 + iota
                a = argp_fb[pl.ds(i * 16, 16)]
                valid = (a < HW) & (q > 0)
                gv = gath_v[pl.ds(i * 16, 16)]
                outb_v[pl.ds(i * 16, 16)] = jnp.where(
                    valid & (gv < 10000.0), gv, 0.0)
                return 0

            lax.fori_loop(0, SHARD // 16, emit, 0)
            pltpu.sync_copy(outb_v, out_hbm.at[pl.ds((b * C + c) * HW + base, SHARD)])
        return 0

    lax.fori_loop(0, B, per_batch, 0)


@jax.jit
def kernel(x, flow_in):
    lin, pvn = _prep(x, flow_in)
    xf = x.reshape(B * C * HW)
    mesh = plsc.VectorSubcoreMesh(core_axis_name="c", subcore_axis_name="s")
    splat = functools.partial(
        pl.kernel,
        mesh=mesh,
        compiler_params=pltpu.CompilerParams(needs_layout_passes=False),
        out_type=jax.ShapeDtypeStruct((B * C * HW,), jnp.float32),
        scratch_types=[
            pltpu.VMEM((CHUNK,), jnp.int32),
            pltpu.VMEM((CHUNK,), jnp.float32),
            pltpu.VMEM((SHARD,), jnp.float32),
            pltpu.VMEM((SHARD,), jnp.int32),
            pltpu.VMEM((SHARD,), jnp.int32),
            pltpu.VMEM((SHARD,), jnp.float32),
            pltpu.VMEM((SHARD,), jnp.float32),
            pltpu.SemaphoreType.DMA,
        ],
    )(_splat_body)
    out = splat(lin, pvn, xf)
    return out.reshape(B, C, H, W)
